# baseline (speedup 1.0000x reference)
import jax
import jax.numpy as jnp
from jax.experimental import pallas as pl
from jax.experimental.pallas import tpu as pltpu


def _bf16(v):
    return v.astype(jnp.bfloat16)


def _xw_kernel(x_ref, w_ref, o_ref):
    o_ref[...] = _bf16(jnp.dot(_bf16(x_ref[...]), w_ref[...],
                               preferred_element_type=jnp.float32))


def _make_gcn_kernel(n, bm, nblk, rb):
    sp = nblk - rb

    def body(adj_ref, s1_ref, b1_ref, b2_ref, b3_ref, w2_ref, w3_ref,
             out_ref, spill_ref,
             res_ref, pp_ref, s2_ref, s3_ref, semw, semr):
        l = pl.program_id(0)
        i = pl.program_id(1)

        cnt = ((i + 1) * rb) // nblk
        is_res = cnt > (i * rb) // nblk
        r_idx = cnt - 1
        sp_idx = i - cnt
        b12 = jnp.where(is_res, r_idx, rb + sp_idx)

        @pl.when(l == 0)
        def _():
            adjb = _bf16(adj_ref[...])
            blk0 = jnp.where(i < sp, i + rb, i - sp)
            acc = jnp.dot(adjb, s1_ref[...], preferred_element_type=jnp.float32)
            h = _bf16(jnp.maximum(acc + b1_ref[...], 0.0))
            s2_ref[pl.ds(blk0 * bm, bm), :] = _bf16(
                jnp.dot(h, w2_ref[...], preferred_element_type=jnp.float32))

            @pl.when(i < sp)
            def _():
                slot = jax.lax.rem(i, 2)

                @pl.when(i >= 2)
                def _():
                    pltpu.make_async_copy(
                        pp_ref.at[slot],
                        spill_ref.at[pl.ds((i - 2) * bm, bm), :],
                        semw.at[slot]).wait()

                pp_ref[slot] = adjb
                pltpu.make_async_copy(
                    pp_ref.at[slot],
                    spill_ref.at[pl.ds(i * bm, bm), :],
                    semw.at[slot]).start()

            @pl.when(i >= sp)
            def _():
                res_ref[pl.ds((i - sp) * bm, bm), :] = adjb

            @pl.when(i == sp)
            def _():
                pltpu.make_async_copy(
                    pp_ref.at[jax.lax.rem(sp - 2, 2)],
                    spill_ref.at[pl.ds((sp - 2) * bm, bm), :],
                    semw.at[jax.lax.rem(sp - 2, 2)]).wait()

            @pl.when(i == sp + 1)
            def _():
                pltpu.make_async_copy(
                    pp_ref.at[jax.lax.rem(sp - 1, 2)],
                    spill_ref.at[pl.ds((sp - 1) * bm, bm), :],
                    semw.at[jax.lax.rem(sp - 1, 2)]).wait()

        j = i + 2
        jw = jax.lax.rem(j, nblk)
        cnt_j = ((jw + 1) * rb) // nblk
        j_res = cnt_j > (jw * rb) // nblk
        sp_j = jw - cnt_j
        slot_j = jax.lax.rem(sp_j, 2)
        do_pf = jnp.logical_and(
            jnp.logical_not(j_res),
            (l == 1) | ((l == 0) & (j >= nblk)) | ((l == 2) & (j < nblk)))

        def prefetch():
            @pl.when(do_pf)
            def _():
                pltpu.make_async_copy(
                    spill_ref.at[pl.ds(sp_j * bm, bm), :],
                    pp_ref.at[slot_j],
                    semr.at[slot_j]).start()

        @pl.when((l == 0) & (i >= nblk - 2))
        def _():
            prefetch()

        @pl.when(l == 1)
        def _():
            def compute(src):
                acc = jnp.dot(src, s2_ref[...],
                              preferred_element_type=jnp.float32)
                h = _bf16(jnp.maximum(acc + b2_ref[...], 0.0))
                s3_ref[pl.ds(b12 * bm, bm), :] = _bf16(
                    jnp.dot(h, w3_ref[...], preferred_element_type=jnp.float32))

            @pl.when(is_res)
            def _():
                compute(res_ref[pl.ds(r_idx * bm, bm), :])

            @pl.when(jnp.logical_not(is_res))
            def _():
                slot = jax.lax.rem(sp_idx, 2)
                pltpu.make_async_copy(
                    spill_ref.at[pl.ds(sp_idx * bm, bm), :],
                    pp_ref.at[slot], semr.at[slot]).wait()
                compute(pp_ref[slot])

            prefetch()

        @pl.when(l == 2)
        def _():
            def compute(src):
                acc = jnp.dot(src, s3_ref[...],
                              preferred_element_type=jnp.float32)
                h = acc + b3_ref[...]
                m = jnp.max(h, axis=1, keepdims=True)
                lse = jnp.log(jnp.sum(jnp.exp(h - m), axis=1,
                                      keepdims=True)) + m
                out_ref[...] = h - lse

            @pl.when(is_res)
            def _():
                compute(res_ref[pl.ds(r_idx * bm, bm), :])

            @pl.when(jnp.logical_not(is_res))
            def _():
                slot = jax.lax.rem(sp_idx, 2)
                pltpu.make_async_copy(
                    spill_ref.at[pl.ds(sp_idx * bm, bm), :],
                    pp_ref.at[slot], semr.at[slot]).wait()
                compute(pp_ref[slot])

            prefetch()

    return body


@jax.jit
def kernel(x, adj, W1, b1, W2, b2, W3, b3):
    n = adj.shape[0]
    nh = W2.shape[0]
    nc = W3.shape[1]
    bm = 80 if n % 80 == 0 else (16 if n % 16 == 0 else n)
    nblk = n // bm
    rb = max(2, min(28, nblk - 2))
    sp = nblk - rb

    w1, w2, w3 = _bf16(W1), _bf16(W2), _bf16(W3)
    s1 = pl.pallas_call(
        _xw_kernel,
        out_shape=jax.ShapeDtypeStruct((n, nh), jnp.bfloat16),
    )(x, w1)
    b1r = b1.reshape(1, -1)
    b2r = b2.reshape(1, -1)
    b3r = b3.reshape(1, -1)

    def res_blk(idx):
        c = ((idx + 1) * rb) // nblk
        return jnp.where(c > (idx * rb) // nblk, c - 1, rb + idx - c)

    out, _ = pl.pallas_call(
        _make_gcn_kernel(n, bm, nblk, rb),
        grid=(3, nblk),
        in_specs=[
            pl.BlockSpec((bm, n),
                         lambda l, i: (jnp.where(
                             l == 0,
                             jnp.where(i < sp, i + rb, i - sp),
                             rb - 1), 0)),
            pl.BlockSpec((n, nh), lambda l, i: (0, 0)),
            pl.BlockSpec((1, nh), lambda l, i: (0, 0)),
            pl.BlockSpec((1, nh), lambda l, i: (0, 0)),
            pl.BlockSpec((1, nc), lambda l, i: (0, 0)),
            pl.BlockSpec((nh, nh), lambda l, i: (0, 0)),
            pl.BlockSpec((nh, nc), lambda l, i: (0, 0)),
        ],
        out_specs=[
            pl.BlockSpec((bm, nc),
                         lambda l, i: (jnp.where(l == 2, res_blk(i), rb), 0)),
            pl.BlockSpec(memory_space=pltpu.MemorySpace.HBM),
        ],
        out_shape=[
            jax.ShapeDtypeStruct((n, nc), jnp.float32),
            jax.ShapeDtypeStruct((sp * bm, n), jnp.bfloat16),
        ],
        scratch_shapes=[
            pltpu.VMEM((rb * bm, n), jnp.bfloat16),
            pltpu.VMEM((2, bm, n), jnp.bfloat16),
            pltpu.VMEM((n, nh), jnp.bfloat16),
            pltpu.VMEM((n, nc), jnp.bfloat16),
            pltpu.SemaphoreType.DMA((2,)),
            pltpu.SemaphoreType.DMA((2,)),
        ],
        compiler_params=pltpu.CompilerParams(
            dimension_semantics=("arbitrary", "arbitrary"),
            vmem_limit_bytes=64 * 1024 * 1024,
        ),
    )(adj, s1, b1r, b2r, b3r, w2, w3)
    return out


# speedup vs baseline: 1.0920x; 1.0920x over previous
import jax
import jax.numpy as jnp
from jax.experimental import pallas as pl
from jax.experimental.pallas import tpu as pltpu


def _bf16(v):
    return v.astype(jnp.bfloat16)


def _xw_kernel(x_ref, w_ref, o_ref):
    o_ref[...] = _bf16(jnp.dot(_bf16(x_ref[...]), w_ref[...],
                               preferred_element_type=jnp.float32))


def _make_gcn_kernel(n, bm, nblk, rb):
    sp = nblk - rb

    def body(adj_ref, s1_ref, b1_ref, b2_ref, b3_ref, w2_ref, w3_ref,
             out_ref, spill_ref,
             res_ref, rd_ref, wr_ref, s2_ref, s3_ref, semw, semr):
        l = pl.program_id(0)
        i = pl.program_id(1)

        cnt = ((i + 1) * rb) // nblk
        is_res = cnt > (i * rb) // nblk
        r_idx = cnt - 1
        sp_idx = i - cnt
        b12 = jnp.where(is_res, r_idx, rb + sp_idx)

        j = i + 3
        jw = jax.lax.rem(j, nblk)
        cnt_j = ((jw + 1) * rb) // nblk
        j_res = cnt_j > (jw * rb) // nblk
        sp_j = jw - cnt_j
        slot_j = jax.lax.rem(sp_j, 4)
        do_pf = jnp.logical_and(
            jnp.logical_not(j_res),
            (l == 1) | ((l == 0) & (j >= nblk)) | ((l == 2) & (j < nblk)))

        def prefetch():
            @pl.when(do_pf)
            def _():
                pltpu.make_async_copy(
                    spill_ref.at[pl.ds(sp_j * bm, bm), :],
                    rd_ref.at[slot_j],
                    semr.at[slot_j]).start()

        @pl.when(l == 0)
        def _():
            adjb = _bf16(adj_ref[...])

            @pl.when(i < sp)
            def _():
                slot = jax.lax.rem(i, 2)

                @pl.when(i >= 2)
                def _():
                    pltpu.make_async_copy(
                        wr_ref.at[slot],
                        spill_ref.at[pl.ds((i - 2) * bm, bm), :],
                        semw.at[slot]).wait()

                wr_ref[slot] = adjb
                pltpu.make_async_copy(
                    wr_ref.at[slot],
                    spill_ref.at[pl.ds(i * bm, bm), :],
                    semw.at[slot]).start()

            @pl.when(i >= sp)
            def _():
                res_ref[pl.ds((i - sp) * bm, bm), :] = adjb

            @pl.when(i == sp)
            def _():
                pltpu.make_async_copy(
                    wr_ref.at[jax.lax.rem(sp - 2, 2)],
                    spill_ref.at[pl.ds((sp - 2) * bm, bm), :],
                    semw.at[jax.lax.rem(sp - 2, 2)]).wait()

            @pl.when(i == sp + 1)
            def _():
                pltpu.make_async_copy(
                    wr_ref.at[jax.lax.rem(sp - 1, 2)],
                    spill_ref.at[pl.ds((sp - 1) * bm, bm), :],
                    semw.at[jax.lax.rem(sp - 1, 2)]).wait()

            prefetch()

            blk0 = jnp.where(i < sp, i + rb, i - sp)
            acc = jnp.dot(adjb, s1_ref[...], preferred_element_type=jnp.float32)
            h = _bf16(jnp.maximum(acc + b1_ref[...], 0.0))
            s2_ref[pl.ds(blk0 * bm, bm), :] = _bf16(
                jnp.dot(h, w2_ref[...], preferred_element_type=jnp.float32))

        @pl.when(l == 1)
        def _():
            def compute(src):
                acc = jnp.dot(src, s2_ref[...],
                              preferred_element_type=jnp.float32)
                h = _bf16(jnp.maximum(acc + b2_ref[...], 0.0))
                s3_ref[pl.ds(b12 * bm, bm), :] = _bf16(
                    jnp.dot(h, w3_ref[...], preferred_element_type=jnp.float32))

            @pl.when(is_res)
            def _():
                prefetch()
                compute(res_ref[pl.ds(r_idx * bm, bm), :])

            @pl.when(jnp.logical_not(is_res))
            def _():
                slot = jax.lax.rem(sp_idx, 4)
                pltpu.make_async_copy(
                    spill_ref.at[pl.ds(sp_idx * bm, bm), :],
                    rd_ref.at[slot], semr.at[slot]).wait()
                prefetch()
                compute(rd_ref[slot])

        @pl.when(l == 2)
        def _():
            def compute(src):
                acc = jnp.dot(src, s3_ref[...],
                              preferred_element_type=jnp.float32)
                h = acc + b3_ref[...]
                m = jnp.max(h, axis=1, keepdims=True)
                lse = jnp.log(jnp.sum(jnp.exp(h - m), axis=1,
                                      keepdims=True)) + m
                out_ref[...] = h - lse

            @pl.when(is_res)
            def _():
                prefetch()
                compute(res_ref[pl.ds(r_idx * bm, bm), :])

            @pl.when(jnp.logical_not(is_res))
            def _():
                slot = jax.lax.rem(sp_idx, 4)
                pltpu.make_async_copy(
                    spill_ref.at[pl.ds(sp_idx * bm, bm), :],
                    rd_ref.at[slot], semr.at[slot]).wait()
                prefetch()
                compute(rd_ref[slot])

    return body


@jax.jit
def kernel(x, adj, W1, b1, W2, b2, W3, b3):
    n = adj.shape[0]
    nh = W2.shape[0]
    nc = W3.shape[1]
    bm = 80 if n % 80 == 0 else (16 if n % 16 == 0 else n)
    nblk = n // bm
    rb = max(2, min(23, nblk - 4))
    sp = nblk - rb

    w1, w2, w3 = _bf16(W1), _bf16(W2), _bf16(W3)
    s1 = pl.pallas_call(
        _xw_kernel,
        out_shape=jax.ShapeDtypeStruct((n, nh), jnp.bfloat16),
    )(x, w1)
    b1r = b1.reshape(1, -1)
    b2r = b2.reshape(1, -1)
    b3r = b3.reshape(1, -1)

    def res_blk(idx):
        c = ((idx + 1) * rb) // nblk
        return jnp.where(c > (idx * rb) // nblk, c - 1, rb + idx - c)

    out, _ = pl.pallas_call(
        _make_gcn_kernel(n, bm, nblk, rb),
        grid=(3, nblk),
        in_specs=[
            pl.BlockSpec((bm, n),
                         lambda l, i: (jnp.where(
                             l == 0,
                             jnp.where(i < sp, i + rb, i - sp),
                             rb - 1), 0)),
            pl.BlockSpec((n, nh), lambda l, i: (0, 0)),
            pl.BlockSpec((1, nh), lambda l, i: (0, 0)),
            pl.BlockSpec((1, nh), lambda l, i: (0, 0)),
            pl.BlockSpec((1, nc), lambda l, i: (0, 0)),
            pl.BlockSpec((nh, nh), lambda l, i: (0, 0)),
            pl.BlockSpec((nh, nc), lambda l, i: (0, 0)),
        ],
        out_specs=[
            pl.BlockSpec((bm, nc),
                         lambda l, i: (jnp.where(l == 2, res_blk(i), rb), 0)),
            pl.BlockSpec(memory_space=pltpu.MemorySpace.HBM),
        ],
        out_shape=[
            jax.ShapeDtypeStruct((n, nc), jnp.float32),
            jax.ShapeDtypeStruct((sp * bm, n), jnp.bfloat16),
        ],
        scratch_shapes=[
            pltpu.VMEM((rb * bm, n), jnp.bfloat16),
            pltpu.VMEM((4, bm, n), jnp.bfloat16),
            pltpu.VMEM((2, bm, n), jnp.bfloat16),
            pltpu.VMEM((n, nh), jnp.bfloat16),
            pltpu.VMEM((n, nc), jnp.bfloat16),
            pltpu.SemaphoreType.DMA((2,)),
            pltpu.SemaphoreType.DMA((4,)),
        ],
        compiler_params=pltpu.CompilerParams(
            dimension_semantics=("arbitrary", "arbitrary"),
            vmem_limit_bytes=64 * 1024 * 1024,
        ),
    )(adj, s1, b1r, b2r, b3r, w2, w3)
    return out


# speedup vs baseline: 1.4443x; 1.3227x over previous
import jax
import jax.numpy as jnp
from jax.experimental import pallas as pl


def _bf16(v):
    return v.astype(jnp.bfloat16)


def _xw_kernel(x_ref, w_ref, o_ref):
    o_ref[...] = _bf16(jnp.dot(_bf16(x_ref[...]), w_ref[...],
                               preferred_element_type=jnp.float32))


def _first_layer_kernel(adj_ref, s_ref, b_ref, w_ref, o_ref, adjb_ref):
    adjb = _bf16(adj_ref[...])
    adjb_ref[...] = adjb
    acc = jnp.dot(adjb, s_ref[...], preferred_element_type=jnp.float32)
    h = _bf16(jnp.maximum(acc + b_ref[...], 0.0))
    o_ref[...] = _bf16(jnp.dot(h, w_ref[...],
                               preferred_element_type=jnp.float32))


def _mid_layer_kernel(adj_ref, s_ref, b_ref, w_ref, o_ref):
    acc = jnp.dot(adj_ref[...], s_ref[...], preferred_element_type=jnp.float32)
    h = _bf16(jnp.maximum(acc + b_ref[...], 0.0))
    o_ref[...] = _bf16(jnp.dot(h, w_ref[...],
                               preferred_element_type=jnp.float32))


def _last_layer_kernel(adj_ref, s_ref, b_ref, o_ref):
    acc = jnp.dot(adj_ref[...], s_ref[...], preferred_element_type=jnp.float32)
    h = acc + b_ref[...]
    m = jnp.max(h, axis=1, keepdims=True)
    lse = jnp.log(jnp.sum(jnp.exp(h - m), axis=1, keepdims=True)) + m
    o_ref[...] = h - lse


def _row_block(n, target):
    for bm in (target, 400, 200, 80, 40, 8):
        if bm <= target and n % bm == 0:
            return bm
    return n


def _layer_call(body, adj, s, b, extra, out_cols, out_dtype, bm_target,
                emit_adj_bf16=False):
    n = adj.shape[0]
    bm = _row_block(n, bm_target)
    grid = (n // bm,)
    k = s.shape[1]
    in_specs = [
        pl.BlockSpec((bm, n), lambda i: (i, 0)),
        pl.BlockSpec((n, k), lambda i: (0, 0)),
        pl.BlockSpec((1, b.shape[1]), lambda i: (0, 0)),
    ]
    args = [adj, s, b]
    if extra is not None:
        in_specs.append(pl.BlockSpec(extra.shape, lambda i: (0, 0)))
        args.append(extra)
    out_specs = pl.BlockSpec((bm, out_cols), lambda i: (i, 0))
    out_shape = jax.ShapeDtypeStruct((n, out_cols), out_dtype)
    if emit_adj_bf16:
        out_specs = [out_specs, pl.BlockSpec((bm, n), lambda i: (i, 0))]
        out_shape = [out_shape, jax.ShapeDtypeStruct((n, n), jnp.bfloat16)]
    return pl.pallas_call(
        body,
        grid=grid,
        in_specs=in_specs,
        out_specs=out_specs,
        out_shape=out_shape,
    )(*args)


@jax.jit
def kernel(x, adj, W1, b1, W2, b2, W3, b3):
    n = adj.shape[0]
    w1, w2, w3 = _bf16(W1), _bf16(W2), _bf16(W3)
    s1 = pl.pallas_call(
        _xw_kernel,
        out_shape=jax.ShapeDtypeStruct((n, W1.shape[1]), jnp.bfloat16),
    )(x, w1)
    b1r = b1.reshape(1, -1)
    b2r = b2.reshape(1, -1)
    b3r = b3.reshape(1, -1)
    s2, adj_bf = _layer_call(_first_layer_kernel, adj, s1, b1r, w2,
                             W2.shape[1], jnp.bfloat16, 400,
                             emit_adj_bf16=True)
    s3 = _layer_call(_mid_layer_kernel, adj_bf, s2, b2r, w3,
                     W3.shape[1], jnp.bfloat16, 400)
    out = _layer_call(_last_layer_kernel, adj_bf, s3, b3r, None,
                      W3.shape[1], jnp.float32, 400)
    return out
